# baseline (speedup 1.0000x reference)
import functools

import jax
import jax.numpy as jnp
from jax import lax
from jax.experimental import pallas as pl
from jax.experimental.pallas import tpu as pltpu
from jax.experimental.pallas import tpu_sc as plsc

NUM_SC_CORES = 2
NUM_SUBCORES = 16
NUM_WORKERS = NUM_SC_CORES * NUM_SUBCORES


def _proj_body(x_ref, w_ref, o_ref):
    o_ref[...] = jnp.dot(x_ref[...], w_ref[...],
                         preferred_element_type=jnp.float32)


def _proj_bias_body(x_ref, w_ref, b_ref, o_ref):
    o_ref[...] = jnp.dot(x_ref[...], w_ref[...],
                         preferred_element_type=jnp.float32) + b_ref[...]


def _matmul(x, w, block_rows):
    n, d = x.shape
    e = w.shape[1]
    return pl.pallas_call(
        _proj_body,
        grid=(n // block_rows,),
        in_specs=[
            pl.BlockSpec((block_rows, d), lambda i: (i, 0)),
            pl.BlockSpec((d, e), lambda i: (0, 0)),
        ],
        out_specs=pl.BlockSpec((block_rows, e), lambda i: (i, 0)),
        out_shape=jax.ShapeDtypeStruct((n, e), jnp.float32),
    )(x, w)


def _matmul_bias(x, w, b2d):
    n, d = x.shape
    e = w.shape[1]
    return pl.pallas_call(
        _proj_bias_body,
        grid=(1,),
        in_specs=[
            pl.BlockSpec((n, d), lambda i: (0, 0)),
            pl.BlockSpec((d, e), lambda i: (0, 0)),
            pl.BlockSpec((1, e), lambda i: (0, 0)),
        ],
        out_specs=pl.BlockSpec((n, e), lambda i: (0, 0)),
        out_shape=jax.ShapeDtypeStruct((n, e), jnp.float32),
    )(x, w, b2d)


def _packproj_body(x_ref, w_ref, r_ref, e_ref, proj_ref, adj_ref):
    proj_ref[...] = jnp.dot(x_ref[...], w_ref[...],
                            preferred_element_type=jnp.float32)
    rows = r_ref.shape[0]
    pad = adj_ref.shape[1] - 2 * r_ref.shape[1]
    adj_ref[...] = jnp.concatenate(
        [r_ref[...], e_ref[...],
         jnp.zeros((rows, pad), jnp.int32)], axis=-1)


def _packproj(entity_embeddings, w2, neighbor_rel, neighbor_ent,
              block_rows, width):
    n, d = entity_embeddings.shape
    e = w2.shape[1]
    m = neighbor_rel.shape[1]
    return pl.pallas_call(
        _packproj_body,
        grid=(n // block_rows,),
        in_specs=[
            pl.BlockSpec((block_rows, d), lambda i: (i, 0)),
            pl.BlockSpec((d, e), lambda i: (0, 0)),
            pl.BlockSpec((block_rows, m), lambda i: (i, 0)),
            pl.BlockSpec((block_rows, m), lambda i: (i, 0)),
        ],
        out_specs=[
            pl.BlockSpec((block_rows, e), lambda i: (i, 0)),
            pl.BlockSpec((block_rows, width), lambda i: (i, 0)),
        ],
        out_shape=[
            jax.ShapeDtypeStruct((n, e), jnp.float32),
            jax.ShapeDtypeStruct((n, width), jnp.int32),
        ],
    )(entity_embeddings, w2, neighbor_rel, neighbor_ent)


def _build_sc_merged(batch, max_n, e_dim, n_rel_pad, cq, ch):
    bq = batch // NUM_WORKERS
    rpt = bq * max_n
    n_conn_chunks = bq // cq
    n_chunks = rpt // ch
    chunks_per_m = bq // ch
    lanes = 16
    mesh = plsc.VectorSubcoreMesh(core_axis_name="c", subcore_axis_name="s",
                                  num_cores=NUM_SC_CORES,
                                  num_subcores=NUM_SUBCORES)

    @functools.partial(
        pl.kernel,
        mesh=mesh,
        out_type=[
            jax.ShapeDtypeStruct((batch * max_n, e_dim), jnp.float32),
            jax.ShapeDtypeStruct((batch, e_dim), jnp.float32),
        ],
        scratch_types=[
            pltpu.VMEM((bq,), jnp.int32),
            pltpu.VMEM((cq, 128), jnp.int32),
            pltpu.VMEM((cq, e_dim), jnp.float32),
            pltpu.VMEM((rpt,), jnp.int32),
            pltpu.VMEM((rpt,), jnp.int32),
            pltpu.VMEM((ch, e_dim), jnp.float32),
            pltpu.VMEM((ch, e_dim), jnp.float32),
            pltpu.VMEM((n_rel_pad, e_dim), jnp.float32),
            pltpu.SemaphoreType.DMA,
            pltpu.SemaphoreType.DMA,
            pltpu.SemaphoreType.DMA,
            pltpu.SemaphoreType.DMA,
        ],
        compiler_params=pltpu.CompilerParams(needs_layout_passes=False),
    )
    def merged_kernel(ids_hbm, adj_hbm, etab_hbm, eproj_hbm, rproj_hbm,
                      agg_out, self_out,
                      idq, cbuf, sbuf, fe, fr,
                      be0, be1, relv, sem0, sem1, semc, sems):
        w = lax.axis_index("s") * NUM_SC_CORES + lax.axis_index("c")
        base = w * bq
        pltpu.sync_copy(rproj_hbm, relv)
        pltpu.sync_copy(ids_hbm.at[pl.ds(base, bq)], idq)

        def conn_step(c, carry):
            off = pl.multiple_of(c * cq, 8)
            cpc = pltpu.async_copy(adj_hbm.at[idq.at[pl.ds(off, cq)]],
                                   cbuf, semc)
            cps = pltpu.async_copy(etab_hbm.at[idq.at[pl.ds(off, cq)]],
                                   sbuf, sems)
            cpc.wait()

            def ext(i, carry2):
                rows = i * lanes + lax.iota(jnp.int32, lanes)
                for m in range(max_n):
                    dst = pl.ds(m * bq + off + i * lanes, lanes)
                    fr[dst] = plsc.load_gather(
                        cbuf, [rows, jnp.full((lanes,), m, jnp.int32)])
                    fe[dst] = plsc.load_gather(
                        cbuf, [rows, jnp.full((lanes,), max_n + m,
                                              jnp.int32)])
                return carry2

            lax.fori_loop(0, cq // lanes, ext, 0)
            cps.wait()
            pltpu.sync_copy(sbuf, self_out.at[pl.ds(base + off, cq)])
            return carry

        lax.fori_loop(0, n_conn_chunks, conn_step, 0)

        def issue(c, be, sem):
            off = pl.multiple_of(c * ch, 8)
            pltpu.async_copy(eproj_hbm.at[fe.at[pl.ds(off, ch)]], be, sem)

        def drain(be, sem):
            pltpu.make_async_copy(eproj_hbm.at[fe.at[pl.ds(0, ch)]],
                                  be, sem).wait()

        def fuse_and_store(c, be):
            src = pl.multiple_of(c * ch, 8)

            def fuse(g, carry2):
                rho_vec = fr[pl.ds(src + g * lanes, lanes)]
                for j in range(lanes):
                    r = g * lanes + j
                    rho = rho_vec[j]
                    for k in range(e_dim // lanes):
                        sl = pl.ds(k * lanes, lanes)
                        be[r, sl] = jnp.maximum(
                            be[r, sl] + relv[rho, sl], 0.0)
                return carry2

            lax.fori_loop(0, ch // lanes, fuse, 0)
            m = c // chunks_per_m
            qoff = (c % chunks_per_m) * ch
            dst = m * batch + base + qoff
            pltpu.sync_copy(be, agg_out.at[pl.ds(dst, ch)])

        issue(0, be0, sem0)

        def step(j, carry):
            c0 = j * 2
            issue(c0 + 1, be1, sem1)
            drain(be0, sem0)
            fuse_and_store(c0, be0)

            @pl.when(c0 + 2 < n_chunks)
            def _():
                issue(c0 + 2, be0, sem0)

            drain(be1, sem1)
            fuse_and_store(c0 + 1, be1)
            return carry

        lax.fori_loop(0, n_chunks // 2, step, 0)

    return merged_kernel


def _att_body(agg_ref, self_ref, wa_ref, ws_ref, bs_ref, o_ref):
    agg = agg_ref[...]
    att = jnp.sum(agg * wa_ref[...][None], axis=-1)
    att = att - jnp.max(att, axis=0, keepdims=True)
    ea = jnp.exp(att)
    alpha = ea / jnp.sum(ea, axis=0, keepdims=True)
    neigh = jnp.sum(agg * alpha[:, :, None], axis=0)
    h = jnp.concatenate([self_ref[...], neigh], axis=-1)
    o_ref[...] = jnp.maximum(
        jnp.dot(h, ws_ref[...], preferred_element_type=jnp.float32)
        + bs_ref[...], 0.0)


def _attention(agg3, self_emb, wa_row, w_self, bs2d, qb):
    m, b, e = agg3.shape
    return pl.pallas_call(
        _att_body,
        grid=(b // qb,),
        in_specs=[
            pl.BlockSpec((m, qb, e), lambda i: (0, i, 0)),
            pl.BlockSpec((qb, e), lambda i: (i, 0)),
            pl.BlockSpec((1, e), lambda i: (0, 0)),
            pl.BlockSpec((2 * e, e), lambda i: (0, 0)),
            pl.BlockSpec((1, e), lambda i: (0, 0)),
        ],
        out_specs=pl.BlockSpec((qb, e), lambda i: (i, 0)),
        out_shape=jax.ShapeDtypeStruct((b, e), jnp.float32),
    )(agg3, self_emb, wa_row, w_self, bs2d)


def kernel(entity_ids, neighbor_rel, neighbor_ent, entity_embeddings,
           relation_embeddings, W_agg, b_agg, W_att, b_att, W_self, b_self):
    batch = entity_ids.shape[0]
    n_ent, e_dim = entity_embeddings.shape
    n_rel, r_dim = relation_embeddings.shape
    max_n = neighbor_rel.shape[1]

    n_rel_pad = (n_rel + 7) // 8 * 8
    rel_pad = jnp.pad(relation_embeddings, ((0, n_rel_pad - n_rel), (0, 0)))
    rel_proj = _matmul_bias(rel_pad, W_agg[:r_dim], b_agg.reshape(1, e_dim))
    ent_proj, adj_packed = _packproj(entity_embeddings, W_agg[r_dim:],
                                     neighbor_rel, neighbor_ent,
                                     block_rows=2000, width=128)

    merged_fn = _build_sc_merged(batch, max_n, e_dim, n_rel_pad,
                                 cq=64, ch=128)
    agg, self_emb = merged_fn(entity_ids, adj_packed, entity_embeddings,
                              ent_proj, rel_proj)

    out = _attention(agg.reshape(max_n, batch, e_dim),
                     self_emb, W_att.reshape(1, e_dim).astype(jnp.float32),
                     W_self, b_self.reshape(1, e_dim), qb=512)
    return out


# speedup vs baseline: 1.3560x; 1.3560x over previous
import functools

import jax
import jax.numpy as jnp
from jax import lax
from jax.experimental import pallas as pl
from jax.experimental.pallas import tpu as pltpu
from jax.experimental.pallas import tpu_sc as plsc

NUM_SC_CORES = 2
NUM_SUBCORES = 16
NUM_WORKERS = NUM_SC_CORES * NUM_SUBCORES


def _proj_body(x_ref, w_ref, o_ref):
    o_ref[...] = jnp.dot(x_ref[...], w_ref[...],
                         preferred_element_type=jnp.float32)


def _proj_bias_body(x_ref, w_ref, b_ref, o_ref):
    o_ref[...] = jnp.dot(x_ref[...], w_ref[...],
                         preferred_element_type=jnp.float32) + b_ref[...]


def _matmul(x, w, block_rows):
    n, d = x.shape
    e = w.shape[1]
    return pl.pallas_call(
        _proj_body,
        grid=(n // block_rows,),
        in_specs=[
            pl.BlockSpec((block_rows, d), lambda i: (i, 0)),
            pl.BlockSpec((d, e), lambda i: (0, 0)),
        ],
        out_specs=pl.BlockSpec((block_rows, e), lambda i: (i, 0)),
        out_shape=jax.ShapeDtypeStruct((n, e), jnp.float32),
    )(x, w)


def _matmul_bias(x, w, b2d):
    n, d = x.shape
    e = w.shape[1]
    return pl.pallas_call(
        _proj_bias_body,
        grid=(1,),
        in_specs=[
            pl.BlockSpec((n, d), lambda i: (0, 0)),
            pl.BlockSpec((d, e), lambda i: (0, 0)),
            pl.BlockSpec((1, e), lambda i: (0, 0)),
        ],
        out_specs=pl.BlockSpec((n, e), lambda i: (0, 0)),
        out_shape=jax.ShapeDtypeStruct((n, e), jnp.float32),
    )(x, w, b2d)


def _packproj_body(x_ref, w_ref, r_ref, e_ref, relx_ref, w1_ref, b_ref,
                   proj_ref, adj_ref, relp_ref):
    proj_ref[...] = jnp.dot(x_ref[...], w_ref[...],
                            preferred_element_type=jnp.float32)
    rows = r_ref.shape[0]
    pad = adj_ref.shape[1] - 2 * r_ref.shape[1]
    adj_ref[...] = jnp.concatenate(
        [r_ref[...], e_ref[...],
         jnp.zeros((rows, pad), jnp.int32)], axis=-1)

    @pl.when(pl.program_id(0) == 0)
    def _():
        relp_ref[...] = jnp.dot(relx_ref[...], w1_ref[...],
                                preferred_element_type=jnp.float32) + b_ref[...]


def _packproj(entity_embeddings, w2, neighbor_rel, neighbor_ent,
              rel_pad, w1, b2d, block_rows, width):
    n, d = entity_embeddings.shape
    e = w2.shape[1]
    m = neighbor_rel.shape[1]
    nr = rel_pad.shape[0]
    return pl.pallas_call(
        _packproj_body,
        grid=(n // block_rows,),
        in_specs=[
            pl.BlockSpec((block_rows, d), lambda i: (i, 0)),
            pl.BlockSpec((d, e), lambda i: (0, 0)),
            pl.BlockSpec((block_rows, m), lambda i: (i, 0)),
            pl.BlockSpec((block_rows, m), lambda i: (i, 0)),
            pl.BlockSpec((nr, d), lambda i: (0, 0)),
            pl.BlockSpec((d, e), lambda i: (0, 0)),
            pl.BlockSpec((1, e), lambda i: (0, 0)),
        ],
        out_specs=[
            pl.BlockSpec((block_rows, e), lambda i: (i, 0)),
            pl.BlockSpec((block_rows, width), lambda i: (i, 0)),
            pl.BlockSpec((nr, e), lambda i: (0, 0)),
        ],
        out_shape=[
            jax.ShapeDtypeStruct((n, e), jnp.float32),
            jax.ShapeDtypeStruct((n, width), jnp.int32),
            jax.ShapeDtypeStruct((nr, e), jnp.float32),
        ],
    )(entity_embeddings, w2, neighbor_rel, neighbor_ent, rel_pad, w1, b2d)


def _build_sc_merged(batch, max_n, e_dim, n_rel_pad, cq, ch):
    bq = batch // NUM_WORKERS
    rpt = bq * max_n
    n_conn_chunks = bq // cq
    n_chunks = rpt // ch
    chunks_per_m = bq // ch
    lanes = 16
    mesh = plsc.VectorSubcoreMesh(core_axis_name="c", subcore_axis_name="s",
                                  num_cores=NUM_SC_CORES,
                                  num_subcores=NUM_SUBCORES)

    @functools.partial(
        pl.kernel,
        mesh=mesh,
        out_type=[
            jax.ShapeDtypeStruct((batch * max_n, e_dim), jnp.float32),
            jax.ShapeDtypeStruct((batch, e_dim), jnp.float32),
        ],
        scratch_types=[
            pltpu.VMEM((bq,), jnp.int32),
            pltpu.VMEM((cq, 128), jnp.int32),
            pltpu.VMEM((cq, e_dim), jnp.float32),
            pltpu.VMEM((rpt,), jnp.int32),
            pltpu.VMEM((rpt,), jnp.int32),
            pltpu.VMEM((ch, e_dim), jnp.float32),
            pltpu.VMEM((ch, e_dim), jnp.float32),
            pltpu.VMEM((ch, e_dim), jnp.float32),
            pltpu.VMEM((ch, e_dim), jnp.float32),
            pltpu.SemaphoreType.DMA,
            pltpu.SemaphoreType.DMA,
            pltpu.SemaphoreType.DMA,
            pltpu.SemaphoreType.DMA,
        ],
        compiler_params=pltpu.CompilerParams(needs_layout_passes=False),
    )
    def merged_kernel(ids_hbm, adj_hbm, etab_hbm, eproj_hbm, rproj_hbm,
                      agg_out, self_out,
                      idq, cbuf, sbuf, fe, fr,
                      be0, be1, br0, br1, sem0, sem1, semc, sems):
        w = lax.axis_index("s") * NUM_SC_CORES + lax.axis_index("c")
        base = w * bq
        pltpu.sync_copy(ids_hbm.at[pl.ds(base, bq)], idq)

        def conn_step(c, carry):
            off = pl.multiple_of(c * cq, 8)
            cpc = pltpu.async_copy(adj_hbm.at[idq.at[pl.ds(off, cq)]],
                                   cbuf, semc)
            cps = pltpu.async_copy(etab_hbm.at[idq.at[pl.ds(off, cq)]],
                                   sbuf, sems)
            cpc.wait()

            def ext(i, carry2):
                rows = i * lanes + lax.iota(jnp.int32, lanes)
                for m in range(max_n):
                    dst = pl.ds(m * bq + off + i * lanes, lanes)
                    fr[dst] = plsc.load_gather(
                        cbuf, [rows, jnp.full((lanes,), m, jnp.int32)])
                    fe[dst] = plsc.load_gather(
                        cbuf, [rows, jnp.full((lanes,), max_n + m,
                                              jnp.int32)])
                return carry2

            lax.fori_loop(0, cq // lanes, ext, 0)
            cps.wait()
            pltpu.sync_copy(sbuf, self_out.at[pl.ds(base + off, cq)])
            return carry

        lax.fori_loop(0, n_conn_chunks, conn_step, 0)

        def issue(c, be, br, sem):
            off = pl.multiple_of(c * ch, 8)
            pltpu.async_copy(eproj_hbm.at[fe.at[pl.ds(off, ch)]], be, sem)
            pltpu.async_copy(rproj_hbm.at[fr.at[pl.ds(off, ch)]], br, sem)

        def drain(be, br, sem):
            pltpu.make_async_copy(eproj_hbm.at[fe.at[pl.ds(0, ch)]],
                                  be, sem).wait()
            pltpu.make_async_copy(rproj_hbm.at[fr.at[pl.ds(0, ch)]],
                                  br, sem).wait()

        def fuse_and_store(c, be, br):
            def fuse(r, carry2):
                for k in range(e_dim // lanes):
                    sl = pl.ds(k * lanes, lanes)
                    be[r, sl] = jnp.maximum(be[r, sl] + br[r, sl], 0.0)
                return carry2

            lax.fori_loop(0, ch, fuse, 0)
            m = c // chunks_per_m
            qoff = (c % chunks_per_m) * ch
            dst = m * batch + base + qoff
            pltpu.sync_copy(be, agg_out.at[pl.ds(dst, ch)])

        issue(0, be0, br0, sem0)

        def step(j, carry):
            c0 = j * 2
            issue(c0 + 1, be1, br1, sem1)
            drain(be0, br0, sem0)
            fuse_and_store(c0, be0, br0)

            @pl.when(c0 + 2 < n_chunks)
            def _():
                issue(c0 + 2, be0, br0, sem0)

            drain(be1, br1, sem1)
            fuse_and_store(c0 + 1, be1, br1)
            return carry

        lax.fori_loop(0, n_chunks // 2, step, 0)

    return merged_kernel


def _att_body(agg_ref, self_ref, wa_ref, ws_ref, bs_ref, o_ref):
    agg = agg_ref[...]
    att = jnp.sum(agg * wa_ref[...][None], axis=-1)
    att = att - jnp.max(att, axis=0, keepdims=True)
    ea = jnp.exp(att)
    alpha = ea / jnp.sum(ea, axis=0, keepdims=True)
    neigh = jnp.sum(agg * alpha[:, :, None], axis=0)
    h = jnp.concatenate([self_ref[...], neigh], axis=-1)
    o_ref[...] = jnp.maximum(
        jnp.dot(h, ws_ref[...], preferred_element_type=jnp.float32)
        + bs_ref[...], 0.0)


def _attention(agg3, self_emb, wa_row, w_self, bs2d, qb):
    m, b, e = agg3.shape
    return pl.pallas_call(
        _att_body,
        grid=(b // qb,),
        in_specs=[
            pl.BlockSpec((m, qb, e), lambda i: (0, i, 0)),
            pl.BlockSpec((qb, e), lambda i: (i, 0)),
            pl.BlockSpec((1, e), lambda i: (0, 0)),
            pl.BlockSpec((2 * e, e), lambda i: (0, 0)),
            pl.BlockSpec((1, e), lambda i: (0, 0)),
        ],
        out_specs=pl.BlockSpec((qb, e), lambda i: (i, 0)),
        out_shape=jax.ShapeDtypeStruct((b, e), jnp.float32),
    )(agg3, self_emb, wa_row, w_self, bs2d)


def kernel(entity_ids, neighbor_rel, neighbor_ent, entity_embeddings,
           relation_embeddings, W_agg, b_agg, W_att, b_att, W_self, b_self):
    batch = entity_ids.shape[0]
    n_ent, e_dim = entity_embeddings.shape
    n_rel, r_dim = relation_embeddings.shape
    max_n = neighbor_rel.shape[1]

    n_rel_pad = (n_rel + 7) // 8 * 8
    rel_pad = jnp.pad(relation_embeddings, ((0, n_rel_pad - n_rel), (0, 0)))
    ent_proj, adj_packed, rel_proj = _packproj(
        entity_embeddings, W_agg[r_dim:], neighbor_rel, neighbor_ent,
        rel_pad, W_agg[:r_dim], b_agg.reshape(1, e_dim),
        block_rows=2000, width=128)

    merged_fn = _build_sc_merged(batch, max_n, e_dim, n_rel_pad,
                                 cq=64, ch=128)
    agg, self_emb = merged_fn(entity_ids, adj_packed, entity_embeddings,
                              ent_proj, rel_proj)

    out = _attention(agg.reshape(max_n, batch, e_dim),
                     self_emb, W_att.reshape(1, e_dim).astype(jnp.float32),
                     W_self, b_self.reshape(1, e_dim), qb=512)
    return out


# speedup vs baseline: 1.3684x; 1.0092x over previous
import functools

import jax
import jax.numpy as jnp
from jax import lax
from jax.experimental import pallas as pl
from jax.experimental.pallas import tpu as pltpu
from jax.experimental.pallas import tpu_sc as plsc

NUM_SC_CORES = 2
NUM_SUBCORES = 16
NUM_WORKERS = NUM_SC_CORES * NUM_SUBCORES


def _proj_body(x_ref, w_ref, o_ref):
    o_ref[...] = jnp.dot(x_ref[...], w_ref[...],
                         preferred_element_type=jnp.float32)


def _proj_bias_body(x_ref, w_ref, b_ref, o_ref):
    o_ref[...] = jnp.dot(x_ref[...], w_ref[...],
                         preferred_element_type=jnp.float32) + b_ref[...]


def _matmul(x, w, block_rows):
    n, d = x.shape
    e = w.shape[1]
    return pl.pallas_call(
        _proj_body,
        grid=(n // block_rows,),
        in_specs=[
            pl.BlockSpec((block_rows, d), lambda i: (i, 0)),
            pl.BlockSpec((d, e), lambda i: (0, 0)),
        ],
        out_specs=pl.BlockSpec((block_rows, e), lambda i: (i, 0)),
        out_shape=jax.ShapeDtypeStruct((n, e), jnp.float32),
    )(x, w)


def _matmul_bias(x, w, b2d):
    n, d = x.shape
    e = w.shape[1]
    return pl.pallas_call(
        _proj_bias_body,
        grid=(1,),
        in_specs=[
            pl.BlockSpec((n, d), lambda i: (0, 0)),
            pl.BlockSpec((d, e), lambda i: (0, 0)),
            pl.BlockSpec((1, e), lambda i: (0, 0)),
        ],
        out_specs=pl.BlockSpec((n, e), lambda i: (0, 0)),
        out_shape=jax.ShapeDtypeStruct((n, e), jnp.float32),
    )(x, w, b2d)


def _packproj_body(x_ref, w_ref, r_ref, e_ref, relx_ref, w1_ref, b_ref,
                   proj_ref, adj_ref, relp_ref):
    proj_ref[...] = jnp.dot(x_ref[...], w_ref[...],
                            preferred_element_type=jnp.float32)
    rows = r_ref.shape[0]
    pad = adj_ref.shape[1] - 2 * r_ref.shape[1]
    adj_ref[...] = jnp.concatenate(
        [r_ref[...], e_ref[...],
         jnp.zeros((rows, pad), jnp.int32)], axis=-1)

    @pl.when(pl.program_id(0) == 0)
    def _():
        relp_ref[...] = jnp.dot(relx_ref[...], w1_ref[...],
                                preferred_element_type=jnp.float32) + b_ref[...]


def _packproj(entity_embeddings, w2, neighbor_rel, neighbor_ent,
              rel_pad, w1, b2d, block_rows, width):
    n, d = entity_embeddings.shape
    e = w2.shape[1]
    m = neighbor_rel.shape[1]
    nr = rel_pad.shape[0]
    return pl.pallas_call(
        _packproj_body,
        grid=(n // block_rows,),
        in_specs=[
            pl.BlockSpec((block_rows, d), lambda i: (i, 0)),
            pl.BlockSpec((d, e), lambda i: (0, 0)),
            pl.BlockSpec((block_rows, m), lambda i: (i, 0)),
            pl.BlockSpec((block_rows, m), lambda i: (i, 0)),
            pl.BlockSpec((nr, d), lambda i: (0, 0)),
            pl.BlockSpec((d, e), lambda i: (0, 0)),
            pl.BlockSpec((1, e), lambda i: (0, 0)),
        ],
        out_specs=[
            pl.BlockSpec((block_rows, e), lambda i: (i, 0)),
            pl.BlockSpec((block_rows, width), lambda i: (i, 0)),
            pl.BlockSpec((nr, e), lambda i: (0, 0)),
        ],
        out_shape=[
            jax.ShapeDtypeStruct((n, e), jnp.float32),
            jax.ShapeDtypeStruct((n, width), jnp.int32),
            jax.ShapeDtypeStruct((nr, e), jnp.float32),
        ],
    )(entity_embeddings, w2, neighbor_rel, neighbor_ent, rel_pad, w1, b2d)


def _build_sc_merged(batch, max_n, e_dim, n_rel_pad, cq, ch):
    bq = batch // NUM_WORKERS
    rpt = bq * max_n
    n_conn_chunks = bq // cq
    n_chunks = rpt // ch
    chunks_per_m = bq // ch
    lanes = 16
    mesh = plsc.VectorSubcoreMesh(core_axis_name="c", subcore_axis_name="s",
                                  num_cores=NUM_SC_CORES,
                                  num_subcores=NUM_SUBCORES)

    @functools.partial(
        pl.kernel,
        mesh=mesh,
        out_type=[
            jax.ShapeDtypeStruct((batch * max_n, e_dim), jnp.float32),
            jax.ShapeDtypeStruct((batch, e_dim), jnp.float32),
        ],
        scratch_types=[
            pltpu.VMEM((bq,), jnp.int32),
            pltpu.VMEM((cq, 128), jnp.int32),
            pltpu.VMEM((cq, e_dim), jnp.float32),
            pltpu.VMEM((rpt,), jnp.int32),
            pltpu.VMEM((rpt,), jnp.int32),
            pltpu.VMEM((ch, e_dim), jnp.float32),
            pltpu.VMEM((ch, e_dim), jnp.float32),
            pltpu.VMEM((ch, e_dim), jnp.float32),
            pltpu.VMEM((ch, e_dim), jnp.float32),
            pltpu.SemaphoreType.DMA,
            pltpu.SemaphoreType.DMA,
            pltpu.SemaphoreType.DMA,
            pltpu.SemaphoreType.DMA,
        ],
        compiler_params=pltpu.CompilerParams(needs_layout_passes=False),
    )
    def merged_kernel(ids_hbm, adj_hbm, etab_hbm, eproj_hbm, rproj_hbm,
                      agg_out, self_out,
                      idq, cbuf, sbuf, fe, fr,
                      be0, be1, br0, br1, sem0, sem1, semc, sems):
        w = lax.axis_index("s") * NUM_SC_CORES + lax.axis_index("c")
        base = w * bq
        pltpu.sync_copy(ids_hbm.at[pl.ds(base, bq)], idq)

        def conn_step(c, carry):
            off = pl.multiple_of(c * cq, 8)
            cpc = pltpu.async_copy(adj_hbm.at[idq.at[pl.ds(off, cq)]],
                                   cbuf, semc)
            cps = pltpu.async_copy(etab_hbm.at[idq.at[pl.ds(off, cq)]],
                                   sbuf, sems)
            cpc.wait()

            def ext(i, carry2):
                rows = i * lanes + lax.iota(jnp.int32, lanes)
                for m in range(max_n):
                    dst = pl.ds(m * bq + off + i * lanes, lanes)
                    fr[dst] = plsc.load_gather(
                        cbuf, [rows, jnp.full((lanes,), m, jnp.int32)])
                    fe[dst] = plsc.load_gather(
                        cbuf, [rows, jnp.full((lanes,), max_n + m,
                                              jnp.int32)])
                return carry2

            lax.fori_loop(0, cq // lanes, ext, 0)
            cps.wait()
            pltpu.sync_copy(sbuf, self_out.at[pl.ds(base + off, cq)])
            return carry

        lax.fori_loop(0, n_conn_chunks, conn_step, 0)

        def issue(c, be, br, sem):
            off = pl.multiple_of(c * ch, 8)
            pltpu.async_copy(eproj_hbm.at[fe.at[pl.ds(off, ch)]], be, sem)
            pltpu.async_copy(rproj_hbm.at[fr.at[pl.ds(off, ch)]], br, sem)

        def drain(be, br, sem):
            pltpu.make_async_copy(eproj_hbm.at[fe.at[pl.ds(0, ch)]],
                                  be, sem).wait()
            pltpu.make_async_copy(rproj_hbm.at[fr.at[pl.ds(0, ch)]],
                                  br, sem).wait()

        def fuse_and_store(c, be, br):
            def fuse(r, carry2):
                for k in range(e_dim // lanes):
                    sl = pl.ds(k * lanes, lanes)
                    be[r, sl] = jnp.maximum(be[r, sl] + br[r, sl], 0.0)
                return carry2

            lax.fori_loop(0, ch, fuse, 0)
            m = c // chunks_per_m
            qoff = (c % chunks_per_m) * ch
            dst = m * batch + base + qoff
            pltpu.sync_copy(be, agg_out.at[pl.ds(dst, ch)])

        issue(0, be0, br0, sem0)

        def step(j, carry):
            c0 = j * 2
            issue(c0 + 1, be1, br1, sem1)
            drain(be0, br0, sem0)
            fuse_and_store(c0, be0, br0)

            @pl.when(c0 + 2 < n_chunks)
            def _():
                issue(c0 + 2, be0, br0, sem0)

            drain(be1, br1, sem1)
            fuse_and_store(c0 + 1, be1, br1)
            return carry

        lax.fori_loop(0, n_chunks // 2, step, 0)

    return merged_kernel


def _att_body(agg_ref, self_ref, wa_ref, ws_ref, bs_ref, o_ref):
    agg = agg_ref[...]
    att = jnp.sum(agg * wa_ref[...][None], axis=-1)
    att = att - jnp.max(att, axis=0, keepdims=True)
    ea = jnp.exp(att)
    alpha = ea / jnp.sum(ea, axis=0, keepdims=True)
    neigh = jnp.sum(agg * alpha[:, :, None], axis=0)
    h = jnp.concatenate([self_ref[...], neigh], axis=-1)
    o_ref[...] = jnp.maximum(
        jnp.dot(h, ws_ref[...], preferred_element_type=jnp.float32)
        + bs_ref[...], 0.0)


def _attention(agg3, self_emb, wa_row, w_self, bs2d, qb):
    m, b, e = agg3.shape
    return pl.pallas_call(
        _att_body,
        grid=(b // qb,),
        in_specs=[
            pl.BlockSpec((m, qb, e), lambda i: (0, i, 0)),
            pl.BlockSpec((qb, e), lambda i: (i, 0)),
            pl.BlockSpec((1, e), lambda i: (0, 0)),
            pl.BlockSpec((2 * e, e), lambda i: (0, 0)),
            pl.BlockSpec((1, e), lambda i: (0, 0)),
        ],
        out_specs=pl.BlockSpec((qb, e), lambda i: (i, 0)),
        out_shape=jax.ShapeDtypeStruct((b, e), jnp.float32),
    )(agg3, self_emb, wa_row, w_self, bs2d)


def kernel(entity_ids, neighbor_rel, neighbor_ent, entity_embeddings,
           relation_embeddings, W_agg, b_agg, W_att, b_att, W_self, b_self):
    batch = entity_ids.shape[0]
    n_ent, e_dim = entity_embeddings.shape
    n_rel, r_dim = relation_embeddings.shape
    max_n = neighbor_rel.shape[1]

    n_rel_pad = (n_rel + 7) // 8 * 8
    rel_pad = jnp.pad(relation_embeddings, ((0, n_rel_pad - n_rel), (0, 0)))
    ent_proj, adj_packed, rel_proj = _packproj(
        entity_embeddings, W_agg[r_dim:], neighbor_rel, neighbor_ent,
        rel_pad, W_agg[:r_dim], b_agg.reshape(1, e_dim),
        block_rows=2000, width=128)

    merged_fn = _build_sc_merged(batch, max_n, e_dim, n_rel_pad,
                                 cq=128, ch=128)
    agg, self_emb = merged_fn(entity_ids, adj_packed, entity_embeddings,
                              ent_proj, rel_proj)

    out = _attention(agg.reshape(max_n, batch, e_dim),
                     self_emb, W_att.reshape(1, e_dim).astype(jnp.float32),
                     W_self, b_self.reshape(1, e_dim), qb=1024)
    return out


# speedup vs baseline: 1.3687x; 1.0002x over previous
import functools

import jax
import jax.numpy as jnp
from jax import lax
from jax.experimental import pallas as pl
from jax.experimental.pallas import tpu as pltpu
from jax.experimental.pallas import tpu_sc as plsc

NUM_SC_CORES = 2
NUM_SUBCORES = 16
NUM_WORKERS = NUM_SC_CORES * NUM_SUBCORES


def _packproj_body(x_ref, w_ref, r_ref, e_ref, relx_ref, w1_ref, b_ref,
                   proj_ref, adj_ref, relp_ref):
    proj_ref[...] = jnp.dot(x_ref[...], w_ref[...],
                            preferred_element_type=jnp.float32)
    rows = r_ref.shape[0]
    pad = adj_ref.shape[1] - 2 * r_ref.shape[1]
    adj_ref[...] = jnp.concatenate(
        [r_ref[...], e_ref[...],
         jnp.zeros((rows, pad), jnp.int32)], axis=-1)

    @pl.when(pl.program_id(0) == 0)
    def _():
        relp_ref[...] = jnp.dot(relx_ref[...], w1_ref[...],
                                preferred_element_type=jnp.float32) + b_ref[...]


def _packproj(entity_embeddings, w2, neighbor_rel, neighbor_ent,
              rel_pad, w1, b2d, block_rows, width):
    n, d = entity_embeddings.shape
    e = w2.shape[1]
    m = neighbor_rel.shape[1]
    nr = rel_pad.shape[0]
    return pl.pallas_call(
        _packproj_body,
        grid=(n // block_rows,),
        in_specs=[
            pl.BlockSpec((block_rows, d), lambda i: (i, 0)),
            pl.BlockSpec((d, e), lambda i: (0, 0)),
            pl.BlockSpec((block_rows, m), lambda i: (i, 0)),
            pl.BlockSpec((block_rows, m), lambda i: (i, 0)),
            pl.BlockSpec((nr, d), lambda i: (0, 0)),
            pl.BlockSpec((d, e), lambda i: (0, 0)),
            pl.BlockSpec((1, e), lambda i: (0, 0)),
        ],
        out_specs=[
            pl.BlockSpec((block_rows, e), lambda i: (i, 0)),
            pl.BlockSpec((block_rows, width), lambda i: (i, 0)),
            pl.BlockSpec((nr, e), lambda i: (0, 0)),
        ],
        out_shape=[
            jax.ShapeDtypeStruct((n, e), jnp.float32),
            jax.ShapeDtypeStruct((n, width), jnp.int32),
            jax.ShapeDtypeStruct((nr, e), jnp.float32),
        ],
    )(entity_embeddings, w2, neighbor_rel, neighbor_ent, rel_pad, w1, b2d)


def _build_sc_merged(batch, max_n, e_dim, cq, ch):
    bq = batch // NUM_WORKERS
    rpt = bq * max_n
    n_conn_chunks = bq // cq
    n_chunks = rpt // ch
    chunks_per_m = bq // ch
    lanes = 16
    mesh = plsc.VectorSubcoreMesh(core_axis_name="c", subcore_axis_name="s",
                                  num_cores=NUM_SC_CORES,
                                  num_subcores=NUM_SUBCORES)

    @functools.partial(
        pl.kernel,
        mesh=mesh,
        out_type=[
            jax.ShapeDtypeStruct((batch * max_n, e_dim), jnp.float32),
            jax.ShapeDtypeStruct((batch, e_dim), jnp.float32),
        ],
        scratch_types=[
            pltpu.VMEM((bq,), jnp.int32),
            pltpu.VMEM((cq, 128), jnp.int32),
            pltpu.VMEM((cq, e_dim), jnp.float32),
            pltpu.VMEM((rpt,), jnp.int32),
            pltpu.VMEM((rpt,), jnp.int32),
            pltpu.VMEM((ch, e_dim), jnp.float32),
            pltpu.VMEM((ch, e_dim), jnp.float32),
            pltpu.VMEM((ch, e_dim), jnp.float32),
            pltpu.VMEM((ch, e_dim), jnp.float32),
            pltpu.SemaphoreType.DMA,
            pltpu.SemaphoreType.DMA,
            pltpu.SemaphoreType.DMA,
            pltpu.SemaphoreType.DMA,
        ],
        compiler_params=pltpu.CompilerParams(needs_layout_passes=False),
    )
    def merged_kernel(ids_hbm, adj_hbm, etab_hbm, eproj_hbm, rproj_hbm,
                      agg_out, self_out,
                      idq, cbuf, sbuf, fe, fr,
                      be0, be1, br0, br1, sem0, sem1, semc, sems):
        w = lax.axis_index("s") * NUM_SC_CORES + lax.axis_index("c")
        base = w * bq
        pltpu.sync_copy(ids_hbm.at[pl.ds(base, bq)], idq)

        def conn_step(c, carry):
            off = pl.multiple_of(c * cq, 8)
            cpc = pltpu.async_copy(adj_hbm.at[idq.at[pl.ds(off, cq)]],
                                   cbuf, semc)
            cps = pltpu.async_copy(etab_hbm.at[idq.at[pl.ds(off, cq)]],
                                   sbuf, sems)
            cpc.wait()

            def ext(i, carry2):
                rows = i * lanes + lax.iota(jnp.int32, lanes)
                for m in range(max_n):
                    dst = pl.ds(m * bq + off + i * lanes, lanes)
                    fr[dst] = plsc.load_gather(
                        cbuf, [rows, jnp.full((lanes,), m, jnp.int32)])
                    fe[dst] = plsc.load_gather(
                        cbuf, [rows, jnp.full((lanes,), max_n + m,
                                              jnp.int32)])
                return carry2

            lax.fori_loop(0, cq // lanes, ext, 0)
            cps.wait()
            pltpu.sync_copy(sbuf, self_out.at[pl.ds(base + off, cq)])
            return carry

        lax.fori_loop(0, n_conn_chunks, conn_step, 0)

        def issue(c, be, br, sem):
            off = pl.multiple_of(c * ch, 8)
            pltpu.async_copy(eproj_hbm.at[fe.at[pl.ds(off, ch)]], be, sem)
            pltpu.async_copy(rproj_hbm.at[fr.at[pl.ds(off, ch)]], br, sem)

        def drain(be, br, sem):
            pltpu.make_async_copy(eproj_hbm.at[fe.at[pl.ds(0, ch)]],
                                  be, sem).wait()
            pltpu.make_async_copy(rproj_hbm.at[fr.at[pl.ds(0, ch)]],
                                  br, sem).wait()

        def fuse_and_store(c, be, br):
            def fuse(r, carry2):
                for k in range(e_dim // lanes):
                    sl = pl.ds(k * lanes, lanes)
                    be[r, sl] = jnp.maximum(be[r, sl] + br[r, sl], 0.0)
                return carry2

            lax.fori_loop(0, ch, fuse, 0)
            m = c // chunks_per_m
            qoff = (c % chunks_per_m) * ch
            dst = m * batch + base + qoff
            pltpu.sync_copy(be, agg_out.at[pl.ds(dst, ch)])

        issue(0, be0, br0, sem0)

        def step(j, carry):
            c0 = j * 2
            issue(c0 + 1, be1, br1, sem1)
            drain(be0, br0, sem0)
            fuse_and_store(c0, be0, br0)

            @pl.when(c0 + 2 < n_chunks)
            def _():
                issue(c0 + 2, be0, br0, sem0)

            drain(be1, br1, sem1)
            fuse_and_store(c0 + 1, be1, br1)
            return carry

        lax.fori_loop(0, n_chunks // 2, step, 0)

    return merged_kernel


def _att_body(agg_ref, self_ref, wa_ref, ws_ref, bs_ref, o_ref):
    agg = agg_ref[...]
    att = jnp.sum(agg * wa_ref[...][None], axis=-1)
    att = att - jnp.max(att, axis=0, keepdims=True)
    ea = jnp.exp(att)
    alpha = ea / jnp.sum(ea, axis=0, keepdims=True)
    neigh = jnp.sum(agg * alpha[:, :, None], axis=0)
    h = jnp.concatenate([self_ref[...], neigh], axis=-1)
    o_ref[...] = jnp.maximum(
        jnp.dot(h, ws_ref[...], preferred_element_type=jnp.float32)
        + bs_ref[...], 0.0)


def _attention(agg3, self_emb, wa_row, w_self, bs2d, qb):
    m, b, e = agg3.shape
    return pl.pallas_call(
        _att_body,
        grid=(b // qb,),
        in_specs=[
            pl.BlockSpec((m, qb, e), lambda i: (0, i, 0)),
            pl.BlockSpec((qb, e), lambda i: (i, 0)),
            pl.BlockSpec((1, e), lambda i: (0, 0)),
            pl.BlockSpec((2 * e, e), lambda i: (0, 0)),
            pl.BlockSpec((1, e), lambda i: (0, 0)),
        ],
        out_specs=pl.BlockSpec((qb, e), lambda i: (i, 0)),
        out_shape=jax.ShapeDtypeStruct((b, e), jnp.float32),
    )(agg3, self_emb, wa_row, w_self, bs2d)


def kernel(entity_ids, neighbor_rel, neighbor_ent, entity_embeddings,
           relation_embeddings, W_agg, b_agg, W_att, b_att, W_self, b_self):
    batch = entity_ids.shape[0]
    n_ent, e_dim = entity_embeddings.shape
    n_rel, r_dim = relation_embeddings.shape
    max_n = neighbor_rel.shape[1]

    n_rel_pad = (n_rel + 7) // 8 * 8
    rel_pad = jnp.pad(relation_embeddings, ((0, n_rel_pad - n_rel), (0, 0)))
    ent_proj, adj_packed, rel_proj = _packproj(
        entity_embeddings, W_agg[r_dim:], neighbor_rel, neighbor_ent,
        rel_pad, W_agg[:r_dim], b_agg.reshape(1, e_dim),
        block_rows=2000, width=128)

    merged_fn = _build_sc_merged(batch, max_n, e_dim, cq=128, ch=128)
    agg, self_emb = merged_fn(entity_ids, adj_packed, entity_embeddings,
                              ent_proj, rel_proj)

    out = _attention(agg.reshape(max_n, batch, e_dim),
                     self_emb, W_att.reshape(1, e_dim).astype(jnp.float32),
                     W_self, b_self.reshape(1, e_dim), qb=1024)
    return out
